# baseline (speedup 1.0000x reference)
import math

import jax
import jax.numpy as jnp
from jax import lax
from jax.experimental import pallas as pl

_B = 262144
_D = 64
_K = 8
_MIN_SCALE = 1e-10
_LOG2PI = math.log(2.0 * math.pi)

_BLK = 8192
_NSTREAM = 4


def _tc_body(*refs):
    x_refs = refs[:_NSTREAM]
    locs_ref, logscales_ref, logcoefs_ref, out_ref = refs[_NSTREAM:]

    locs = locs_ref[...]
    logscales = logscales_ref[...]
    logcoefs = logcoefs_ref[...]

    scale = jnp.exp(logscales) + _MIN_SCALE
    inv2 = 1.0 / (scale * scale)
    a = -0.5 * inv2
    t = locs * inv2
    lc = logcoefs[0] - jax.nn.logsumexp(logcoefs[0])
    c = (lc
         - jnp.sum(jnp.log(scale), axis=1)
         - 0.5 * _D * _LOG2PI
         - 0.5 * jnp.sum(locs * locs * inv2, axis=1))

    nt = (((1,), (1,)), ((), ()))
    for j, x_ref in enumerate(x_refs):
        x = x_ref[...]
        lp = (lax.dot_general(t, x, nt, preferred_element_type=jnp.float32)
              + lax.dot_general(a, x * x, nt, preferred_element_type=jnp.float32)
              + c[:, None])
        m = jnp.max(lp, axis=0)
        s = jnp.sum(jnp.exp(lp - m[None, :]), axis=0)
        out_ref[0, j, :] = m + jnp.log(s)


@jax.jit
def kernel(x, locs, logscales, logcoefs):
    grid = _B // (_BLK * _NSTREAM)
    x_specs = [
        pl.BlockSpec((_BLK, _D), lambda i, j=j: (i * _NSTREAM + j, 0))
        for j in range(_NSTREAM)
    ]
    out2d = pl.pallas_call(
        _tc_body,
        grid=(grid,),
        in_specs=x_specs + [
            pl.BlockSpec((_K, _D), lambda i: (0, 0)),
            pl.BlockSpec((_K, _D), lambda i: (0, 0)),
            pl.BlockSpec((1, _K), lambda i: (0, 0)),
        ],
        out_specs=pl.BlockSpec((1, _NSTREAM, _BLK), lambda i: (i, 0, 0)),
        out_shape=jax.ShapeDtypeStruct((grid, _NSTREAM, _BLK), jnp.float32),
    )(*([x] * _NSTREAM), locs, logscales, logcoefs)
    return out2d.reshape(_B)
